# baseline (speedup 1.0000x reference)
import functools

import jax
import jax.numpy as jnp
from jax import lax
from jax.experimental import pallas as pl
from jax.experimental.pallas import tpu as pltpu
from jax.experimental.pallas import tpu_sc as plsc

B = 16384
K = 32
PACK = 128 // K
NC = 2
NS = 16
NW = NC * NS
BPW = B // NW


def _sc_gather(ublk, iblk, uidx, iidx, W128, H128, ubf, ibf):
    mesh = plsc.VectorSubcoreMesh(core_axis_name="c", subcore_axis_name="s")

    @functools.partial(
        pl.kernel,
        out_type=(
            jax.ShapeDtypeStruct((B, 128), jnp.float32),
            jax.ShapeDtypeStruct((B, 128), jnp.float32),
            jax.ShapeDtypeStruct((B,), jnp.float32),
            jax.ShapeDtypeStruct((B,), jnp.float32),
        ),
        mesh=mesh,
        scratch_types=[
            pltpu.VMEM((BPW,), jnp.int32),
            pltpu.VMEM((BPW,), jnp.int32),
            pltpu.VMEM((BPW,), jnp.int32),
            pltpu.VMEM((BPW,), jnp.int32),
            pltpu.VMEM((BPW, 128), jnp.float32),
            pltpu.VMEM((BPW,), jnp.float32),
            pltpu.VMEM((BPW,), jnp.float32),
            pltpu.SemaphoreType.DMA,
            pltpu.SemaphoreType.DMA,
        ],
    )
    def gather_kernel(ublk_hbm, iblk_hbm, uidx_hbm, iidx_hbm,
                      w_hbm, h_hbm, ub_hbm, ib_hbm,
                      uo_hbm, vo_hbm, ubo_hbm, ibo_hbm,
                      ublk_v, iblk_v, uidx_v, iidx_v,
                      rows_v, ubg_v, ibg_v, sem, bsem):
        wid = lax.axis_index("s") * NC + lax.axis_index("c")
        base = wid * BPW
        pltpu.sync_copy(ublk_hbm.at[pl.ds(base, BPW)], ublk_v)
        pltpu.sync_copy(iblk_hbm.at[pl.ds(base, BPW)], iblk_v)
        pltpu.sync_copy(uidx_hbm.at[pl.ds(base, BPW)], uidx_v)
        pltpu.sync_copy(iidx_hbm.at[pl.ds(base, BPW)], iidx_v)
        bcps = (
            pltpu.async_copy(ub_hbm.at[uidx_v], ubg_v, bsem),
            pltpu.async_copy(ib_hbm.at[iidx_v], ibg_v, bsem),
        )
        pltpu.async_copy(w_hbm.at[ublk_v], rows_v, sem).wait()
        pltpu.sync_copy(rows_v, uo_hbm.at[pl.ds(base, BPW)])
        pltpu.async_copy(h_hbm.at[iblk_v], rows_v, sem).wait()
        pltpu.sync_copy(rows_v, vo_hbm.at[pl.ds(base, BPW)])
        for cp in bcps:
            cp.wait()
        pltpu.sync_copy(ubg_v, ubo_hbm.at[pl.ds(base, BPW)])
        pltpu.sync_copy(ibg_v, ibo_hbm.at[pl.ds(base, BPW)])

    return gather_kernel(ublk, iblk, uidx, iidx, W128, H128, ubf, ibf)


def _mlp_body(u_ref, v_ref, uo_ref, io_ref, ub_ref, ib_ref,
              w1a_ref, w1b_ref, b1_ref, w2_ref, o_ref):
    uo = uo_ref[...]
    io = io_ref[...]
    blk = uo.shape[0]
    u32 = jnp.zeros((blk, K), jnp.float32)
    v32 = jnp.zeros((blk, K), jnp.float32)
    for t in range(PACK):
        u32 = u32 + jnp.where(uo == t, 1.0, 0.0) * u_ref[:, K * t:K * (t + 1)]
        v32 = v32 + jnp.where(io == t, 1.0, 0.0) * v_ref[:, K * t:K * (t + 1)]
    h = (
        jnp.dot(u32, w1a_ref[...], preferred_element_type=jnp.float32)
        + jnp.dot(v32, w1b_ref[...], preferred_element_type=jnp.float32)
        + b1_ref[...]
    )
    h = jnp.maximum(h, 0.0)
    acc = jnp.sum(h * w2_ref[...], axis=1, keepdims=True)
    o_ref[...] = acc + ub_ref[...] + ib_ref[...]


def _tc_mlp(U128, V128, uo, io, ubg, ibg, w1a, w1b, b1r, w2r):
    blk = 2048
    grid = (B // blk,)
    return pl.pallas_call(
        _mlp_body,
        grid=grid,
        in_specs=[
            pl.BlockSpec((blk, 128), lambda i: (i, 0)),
            pl.BlockSpec((blk, 128), lambda i: (i, 0)),
            pl.BlockSpec((blk, 1), lambda i: (i, 0)),
            pl.BlockSpec((blk, 1), lambda i: (i, 0)),
            pl.BlockSpec((blk, 1), lambda i: (i, 0)),
            pl.BlockSpec((blk, 1), lambda i: (i, 0)),
            pl.BlockSpec((K, K), lambda i: (0, 0)),
            pl.BlockSpec((K, K), lambda i: (0, 0)),
            pl.BlockSpec((1, K), lambda i: (0, 0)),
            pl.BlockSpec((1, K), lambda i: (0, 0)),
        ],
        out_specs=pl.BlockSpec((blk, 1), lambda i: (i, 0)),
        out_shape=jax.ShapeDtypeStruct((B, 1), jnp.float32),
    )(U128, V128, uo, io, ubg, ibg, w1a, w1b, b1r, w2r)


def kernel(x, W, H, W_pre, H_pre, W_eps, H_eps, W1, b1, W2, ub, ib):
    uidx = x[:, 0]
    iidx = x[:, 1]
    W128 = W.reshape(-1, 128)
    H128 = H.reshape(-1, 128)
    U128, V128, ubg, ibg = _sc_gather(
        uidx >> 2, iidx >> 2, uidx, iidx, W128, H128,
        ub.reshape(-1), ib.reshape(-1))
    w1a = W1[:, :K].T
    w1b = W1[:, K:].T
    return _tc_mlp(U128, V128,
                   (uidx & 3).reshape(B, 1), (iidx & 3).reshape(B, 1),
                   ubg.reshape(B, 1), ibg.reshape(B, 1),
                   w1a, w1b, b1.reshape(1, K), W2)


# speedup vs baseline: 2.4116x; 2.4116x over previous
import functools

import jax
import jax.numpy as jnp
from jax import lax
from jax.experimental import pallas as pl
from jax.experimental.pallas import tpu as pltpu
from jax.experimental.pallas import tpu_sc as plsc

B = 16384
K = 32
NC = 2
NS = 16
NW = NC * NS
BPW = B // NW
CHUNK = 128
QUAD = 4


def _sc_gather(uidx, iidx, Wt, Ht, ubf, ibf):
    mesh = plsc.VectorSubcoreMesh(core_axis_name="c", subcore_axis_name="s")

    @functools.partial(
        pl.kernel,
        out_type=(
            jax.ShapeDtypeStruct((K, B), jnp.float32),
            jax.ShapeDtypeStruct((K, B), jnp.float32),
            jax.ShapeDtypeStruct((B,), jnp.float32),
            jax.ShapeDtypeStruct((B,), jnp.float32),
        ),
        mesh=mesh,
        scratch_types=[
            pltpu.VMEM((BPW + 16,), jnp.int32),
            pltpu.VMEM((BPW + 16,), jnp.int32),
            [pltpu.VMEM((K, 128), jnp.float32) for _ in range(QUAD)],
            [pltpu.VMEM((K, 128), jnp.float32) for _ in range(QUAD)],
            pltpu.VMEM((K, CHUNK), jnp.float32),
            pltpu.VMEM((K, CHUNK), jnp.float32),
            pltpu.VMEM((BPW,), jnp.float32),
            pltpu.VMEM((BPW,), jnp.float32),
            pltpu.SemaphoreType.DMA,
            pltpu.SemaphoreType.DMA,
            pltpu.SemaphoreType.DMA,
        ],
        compiler_params=pltpu.CompilerParams(needs_layout_passes=False),
    )
    def gather_kernel(uidx_hbm, iidx_hbm, wt_hbm, ht_hbm, ub_hbm, ib_hbm,
                      ut_hbm, vt_hbm, ubo_hbm, ibo_hbm,
                      uidx_v, iidx_v, wslots, hslots, uo_v, vo_v,
                      ubg_v, ibg_v, wsem, hsem, bsem):
        wid = lax.axis_index("s") * NC + lax.axis_index("c")
        base = wid * BPW
        pltpu.sync_copy(uidx_hbm.at[pl.ds(base, BPW)], uidx_v.at[pl.ds(0, BPW)])
        pltpu.sync_copy(iidx_hbm.at[pl.ds(base, BPW)], iidx_v.at[pl.ds(0, BPW)])
        bcps = (
            pltpu.async_copy(ub_hbm.at[uidx_v.at[pl.ds(0, BPW)]], ubg_v, bsem),
            pltpu.async_copy(ib_hbm.at[iidx_v.at[pl.ds(0, BPW)]], ibg_v, bsem),
        )

        klo = lax.iota(jnp.int32, 16)
        khi = klo + 16

        def fire(tbl_hbm, idx_vec, lane, slots, sem):
            col = idx_vec[lane]
            off = pl.multiple_of((col >> 7) << 7, 128)
            return pltpu.async_copy(
                tbl_hbm.at[:, pl.ds(off, 128)], slots[lane & (QUAD - 1)], sem)

        def extract(idx_vec, lane, slots, out_ref, jc):
            col = idx_vec[lane]
            c = jnp.broadcast_to(col & 127, (16,))
            jcv = jnp.broadcast_to(jc, (16,))
            slot = slots[lane & (QUAD - 1)]
            lo = plsc.load_gather(slot, [klo, c])
            hi = plsc.load_gather(slot, [khi, c])
            plsc.store_scatter(out_ref, [klo, jcv], lo)
            plsc.store_scatter(out_ref, [khi, jcv], hi)

        for ch in range(BPW // CHUNK):
            chbase = ch * CHUNK

            def batch(b, _):
                sbase = chbase + b * QUAD
                uvec = uidx_v[pl.ds(sbase, 16)]
                ivec = iidx_v[pl.ds(sbase, 16)]
                cps = []
                for s in range(QUAD):
                    cps.append(fire(wt_hbm, uvec, s, wslots, wsem))
                    cps.append(fire(ht_hbm, ivec, s, hslots, hsem))
                for cp in cps:
                    cp.wait()
                for s in range(QUAD):
                    extract(uvec, s, wslots, uo_v, b * QUAD + s)
                    extract(ivec, s, hslots, vo_v, b * QUAD + s)
                return 0

            lax.fori_loop(0, CHUNK // QUAD, batch, 0, unroll=False)
            pltpu.sync_copy(uo_v, ut_hbm.at[:, pl.ds(base + chbase, CHUNK)])
            pltpu.sync_copy(vo_v, vt_hbm.at[:, pl.ds(base + chbase, CHUNK)])

        for cp in bcps:
            cp.wait()
        pltpu.sync_copy(ubg_v, ubo_hbm.at[pl.ds(base, BPW)])
        pltpu.sync_copy(ibg_v, ibo_hbm.at[pl.ds(base, BPW)])

    return gather_kernel(uidx, iidx, Wt, Ht, ubf, ibf)


def _mlp_body(ut_ref, vt_ref, ub_ref, ib_ref, w1a_ref, w1b_ref, b1_ref,
              w2_ref, o_ref):
    h = (
        jnp.dot(w1a_ref[...], ut_ref[...], preferred_element_type=jnp.float32)
        + jnp.dot(w1b_ref[...], vt_ref[...], preferred_element_type=jnp.float32)
        + b1_ref[...]
    )
    h = jnp.maximum(h, 0.0)
    acc = jnp.sum(h * w2_ref[...], axis=0, keepdims=True)
    o_ref[...] = acc + ub_ref[...] + ib_ref[...]


def _tc_mlp(Ut, Vt, ubg, ibg, w1a, w1b, b1c, w2c):
    blk = 2048
    grid = (B // blk,)
    return pl.pallas_call(
        _mlp_body,
        grid=grid,
        in_specs=[
            pl.BlockSpec((K, blk), lambda i: (0, i)),
            pl.BlockSpec((K, blk), lambda i: (0, i)),
            pl.BlockSpec((1, blk), lambda i: (0, i)),
            pl.BlockSpec((1, blk), lambda i: (0, i)),
            pl.BlockSpec((K, K), lambda i: (0, 0)),
            pl.BlockSpec((K, K), lambda i: (0, 0)),
            pl.BlockSpec((K, 1), lambda i: (0, 0)),
            pl.BlockSpec((K, 1), lambda i: (0, 0)),
        ],
        out_specs=pl.BlockSpec((1, blk), lambda i: (0, i)),
        out_shape=jax.ShapeDtypeStruct((1, B), jnp.float32),
    )(Ut, Vt, ubg, ibg, w1a, w1b, b1c, w2c)


def kernel(x, W, H, W_pre, H_pre, W_eps, H_eps, W1, b1, W2, ub, ib):
    uidx = x[:, 0]
    iidx = x[:, 1]
    Ut, Vt, ubg, ibg = _sc_gather(uidx, iidx, W.T, H.T,
                                  ub.reshape(-1), ib.reshape(-1))
    w1a = W1[:, :K]
    w1b = W1[:, K:]
    outT = _tc_mlp(Ut, Vt, ubg.reshape(1, B), ibg.reshape(1, B),
                   w1a, w1b, b1.reshape(K, 1), W2.reshape(K, 1))
    return outT.reshape(B, 1)


# speedup vs baseline: 2.9901x; 1.2398x over previous
import functools

import jax
import jax.numpy as jnp
from jax import lax
from jax.experimental import pallas as pl
from jax.experimental.pallas import tpu as pltpu
from jax.experimental.pallas import tpu_sc as plsc

B = 16384
K = 32
NC = 2
NS = 16
NW = NC * NS
BPW = B // NW
CHUNK = 128
QUAD = 4


def _sc_gather(uidx, iidx, Wt, Ht, ubf, ibf):
    mesh = plsc.VectorSubcoreMesh(core_axis_name="c", subcore_axis_name="s")

    @functools.partial(
        pl.kernel,
        out_type=(
            jax.ShapeDtypeStruct((K, B), jnp.float32),
            jax.ShapeDtypeStruct((K, B), jnp.float32),
            jax.ShapeDtypeStruct((B,), jnp.float32),
            jax.ShapeDtypeStruct((B,), jnp.float32),
        ),
        mesh=mesh,
        scratch_types=[
            pltpu.VMEM((BPW + 16,), jnp.int32),
            pltpu.VMEM((BPW + 16,), jnp.int32),
            [pltpu.VMEM((K, 128), jnp.float32) for _ in range(QUAD)],
            [pltpu.VMEM((K, 128), jnp.float32) for _ in range(QUAD)],
            pltpu.VMEM((K, CHUNK), jnp.float32),
            pltpu.VMEM((K, CHUNK), jnp.float32),
            pltpu.VMEM((BPW,), jnp.float32),
            pltpu.VMEM((BPW,), jnp.float32),
            pltpu.SemaphoreType.DMA,
            pltpu.SemaphoreType.DMA,
            pltpu.SemaphoreType.DMA,
        ],
        compiler_params=pltpu.CompilerParams(needs_layout_passes=False),
    )
    def gather_kernel(uidx_hbm, iidx_hbm, wt_hbm, ht_hbm, ub_hbm, ib_hbm,
                      ut_hbm, vt_hbm, ubo_hbm, ibo_hbm,
                      uidx_v, iidx_v, wslots, hslots, uo_v, vo_v,
                      ubg_v, ibg_v, wsem, hsem, bsem):
        wid = lax.axis_index("s") * NC + lax.axis_index("c")
        base = wid * BPW
        pltpu.sync_copy(uidx_hbm.at[pl.ds(base, BPW)], uidx_v.at[pl.ds(0, BPW)])
        pltpu.sync_copy(iidx_hbm.at[pl.ds(base, BPW)], iidx_v.at[pl.ds(0, BPW)])
        bcps = (
            pltpu.async_copy(ub_hbm.at[uidx_v.at[pl.ds(0, BPW)]], ubg_v, bsem),
            pltpu.async_copy(ib_hbm.at[iidx_v.at[pl.ds(0, BPW)]], ibg_v, bsem),
        )

        klo = lax.iota(jnp.int32, 16)
        khi = klo + 16

        def fire(tbl_hbm, idx_vec, lane, slots, sem):
            col = idx_vec[lane]
            off = pl.multiple_of((col >> 7) << 7, 128)
            return pltpu.async_copy(
                tbl_hbm.at[:, pl.ds(off, 128)], slots[lane & (QUAD - 1)], sem)

        def extract(idx_vec, lane, slots, out_ref, jc):
            col = idx_vec[lane]
            c = jnp.broadcast_to(col & 127, (16,))
            jcv = jnp.broadcast_to(jc, (16,))
            slot = slots[lane & (QUAD - 1)]
            lo = plsc.load_gather(slot, [klo, c])
            hi = plsc.load_gather(slot, [khi, c])
            plsc.store_scatter(out_ref, [klo, jcv], lo)
            plsc.store_scatter(out_ref, [khi, jcv], hi)

        for ch in range(BPW // CHUNK):
            chbase = ch * CHUNK

            def batch(b, _):
                sbase = chbase + b * QUAD
                uvec = uidx_v[pl.ds(sbase, 16)]
                ivec = iidx_v[pl.ds(sbase, 16)]
                cps = []
                for s in range(QUAD):
                    cps.append(fire(wt_hbm, uvec, s, wslots, wsem))
                    cps.append(fire(ht_hbm, ivec, s, hslots, hsem))
                for cp in cps:
                    cp.wait()
                for s in range(QUAD):
                    extract(uvec, s, wslots, uo_v, b * QUAD + s)
                    extract(ivec, s, hslots, vo_v, b * QUAD + s)
                return 0

            lax.fori_loop(0, CHUNK // QUAD, batch, 0, unroll=False)
            pltpu.sync_copy(uo_v, ut_hbm.at[:, pl.ds(base + chbase, CHUNK)])
            pltpu.sync_copy(vo_v, vt_hbm.at[:, pl.ds(base + chbase, CHUNK)])

        for cp in bcps:
            cp.wait()
        pltpu.sync_copy(ubg_v, ubo_hbm.at[pl.ds(base, BPW)])
        pltpu.sync_copy(ibg_v, ibo_hbm.at[pl.ds(base, BPW)])

    return gather_kernel(uidx, iidx, Wt, Ht, ubf, ibf)


def _flat_body(u_ref, i_ref, uo_ref, io_ref):
    uo_ref[...] = u_ref[0, :]
    io_ref[...] = i_ref[0, :]


def _tc_flatten(ubT, ibT):
    n = ubT.shape[1]
    blk = 131072
    grid = (pl.cdiv(n, blk),)
    return pl.pallas_call(
        _flat_body,
        grid=grid,
        in_specs=[
            pl.BlockSpec((1, blk), lambda i: (0, i)),
            pl.BlockSpec((1, blk), lambda i: (0, i)),
        ],
        out_specs=[
            pl.BlockSpec((blk,), lambda i: (i,)),
            pl.BlockSpec((blk,), lambda i: (i,)),
        ],
        out_shape=[
            jax.ShapeDtypeStruct((n,), jnp.float32),
            jax.ShapeDtypeStruct((n,), jnp.float32),
        ],
    )(ubT, ibT)


def _mlp_body(ut_ref, vt_ref, ub_ref, ib_ref, w1a_ref, w1b_ref, b1_ref,
              w2_ref, o_ref):
    h = (
        jnp.dot(w1a_ref[...], ut_ref[...], preferred_element_type=jnp.float32)
        + jnp.dot(w1b_ref[...], vt_ref[...], preferred_element_type=jnp.float32)
        + b1_ref[...]
    )
    h = jnp.maximum(h, 0.0)
    acc = jnp.sum(h * w2_ref[...], axis=0, keepdims=True)
    o_ref[...] = acc + ub_ref[...] + ib_ref[...]


def _tc_mlp(Ut, Vt, ubg, ibg, w1a, w1b, b1c, w2c):
    blk = 2048
    grid = (B // blk,)
    return pl.pallas_call(
        _mlp_body,
        grid=grid,
        in_specs=[
            pl.BlockSpec((K, blk), lambda i: (0, i)),
            pl.BlockSpec((K, blk), lambda i: (0, i)),
            pl.BlockSpec((1, blk), lambda i: (0, i)),
            pl.BlockSpec((1, blk), lambda i: (0, i)),
            pl.BlockSpec((K, K), lambda i: (0, 0)),
            pl.BlockSpec((K, K), lambda i: (0, 0)),
            pl.BlockSpec((K, 1), lambda i: (0, 0)),
            pl.BlockSpec((K, 1), lambda i: (0, 0)),
        ],
        out_specs=pl.BlockSpec((1, blk), lambda i: (0, i)),
        out_shape=jax.ShapeDtypeStruct((1, B), jnp.float32),
    )(Ut, Vt, ubg, ibg, w1a, w1b, b1c, w2c)


def kernel(x, W, H, W_pre, H_pre, W_eps, H_eps, W1, b1, W2, ub, ib):
    uidx = x[:, 0]
    iidx = x[:, 1]
    ubf, ibf = _tc_flatten(ub.T, ib.T)
    Ut, Vt, ubg, ibg = _sc_gather(uidx, iidx, W.T, H.T, ubf, ibf)
    w1a = W1[:, :K]
    w1b = W1[:, K:]
    outT = _tc_mlp(Ut, Vt, ubg.reshape(1, B), ibg.reshape(1, B),
                   w1a, w1b, b1.reshape(K, 1), W2.reshape(K, 1))
    return outT.reshape(B, 1)


# speedup vs baseline: 3.7243x; 1.2456x over previous
import functools

import jax
import jax.numpy as jnp
from jax import lax
from jax.experimental import pallas as pl
from jax.experimental.pallas import tpu as pltpu
from jax.experimental.pallas import tpu_sc as plsc

B = 16384
K = 32
NC = 2
NS = 16
NW = NC * NS
BPW = B // NW
CHUNK = 128
QUAD = 4


def _sc_gather(uidx, iidx, Wt, Ht, ubf, ibf):
    mesh = plsc.VectorSubcoreMesh(core_axis_name="c", subcore_axis_name="s")

    @functools.partial(
        pl.kernel,
        out_type=(
            jax.ShapeDtypeStruct((K, B), jnp.float32),
            jax.ShapeDtypeStruct((K, B), jnp.float32),
            jax.ShapeDtypeStruct((B,), jnp.float32),
            jax.ShapeDtypeStruct((B,), jnp.float32),
        ),
        mesh=mesh,
        scratch_types=[
            pltpu.VMEM((BPW + 16,), jnp.int32),
            pltpu.VMEM((BPW + 16,), jnp.int32),
            [pltpu.VMEM((K, 128), jnp.float32) for _ in range(2 * QUAD)],
            [pltpu.VMEM((K, 128), jnp.float32) for _ in range(2 * QUAD)],
            pltpu.VMEM((K, CHUNK), jnp.float32),
            pltpu.VMEM((K, CHUNK), jnp.float32),
            pltpu.VMEM((BPW,), jnp.float32),
            pltpu.VMEM((BPW,), jnp.float32),
            [pltpu.SemaphoreType.DMA for _ in range(4)],
            pltpu.SemaphoreType.DMA,
        ],
        compiler_params=pltpu.CompilerParams(needs_layout_passes=False),
    )
    def gather_kernel(uidx_hbm, iidx_hbm, wt_hbm, ht_hbm, ub_hbm, ib_hbm,
                      ut_hbm, vt_hbm, ubo_hbm, ibo_hbm,
                      uidx_v, iidx_v, wslots, hslots, uo_v, vo_v,
                      ubg_v, ibg_v, sems, bsem):
        wid = lax.axis_index("s") * NC + lax.axis_index("c")
        base = wid * BPW
        pltpu.sync_copy(uidx_hbm.at[pl.ds(base, BPW)], uidx_v.at[pl.ds(0, BPW)])
        pltpu.sync_copy(iidx_hbm.at[pl.ds(base, BPW)], iidx_v.at[pl.ds(0, BPW)])
        bcps = (
            pltpu.async_copy(ub_hbm.at[uidx_v.at[pl.ds(0, BPW)]], ubg_v, bsem),
            pltpu.async_copy(ib_hbm.at[iidx_v.at[pl.ds(0, BPW)]], ibg_v, bsem),
        )

        klo = lax.iota(jnp.int32, 16)
        khi = klo + 16

        def fire_one(tbl_hbm, idx_vec, lane, slot, sem):
            col = idx_vec[lane]
            off = pl.multiple_of((col >> 7) << 7, 128)
            return pltpu.async_copy(tbl_hbm.at[:, pl.ds(off, 128)], slot, sem)

        def fire_batch(sbase, par):
            uvec = uidx_v[pl.ds(sbase, 16)]
            ivec = iidx_v[pl.ds(sbase, 16)]
            for s in range(QUAD):
                fire_one(wt_hbm, uvec, s, wslots[par * QUAD + s], sems[par])
                fire_one(ht_hbm, ivec, s, hslots[par * QUAD + s], sems[2 + par])

        def drain_batch(sbase, par):
            for s in range(QUAD):
                pltpu.make_async_copy(
                    wt_hbm.at[:, pl.ds(0, 128)], wslots[par * QUAD + s],
                    sems[par]).wait()
                pltpu.make_async_copy(
                    ht_hbm.at[:, pl.ds(0, 128)], hslots[par * QUAD + s],
                    sems[2 + par]).wait()

        def extract_one(idx_vec, lane, slot, out_ref, jc):
            col = idx_vec[lane]
            c = jnp.broadcast_to(col & 127, (16,))
            jcv = jnp.broadcast_to(jc, (16,))
            lo = plsc.load_gather(slot, [klo, c])
            hi = plsc.load_gather(slot, [khi, c])
            plsc.store_scatter(out_ref, [klo, jcv], lo)
            plsc.store_scatter(out_ref, [khi, jcv], hi)

        def extract_batch(sbase, par, jcbase):
            uvec = uidx_v[pl.ds(sbase, 16)]
            ivec = iidx_v[pl.ds(sbase, 16)]
            for s in range(QUAD):
                extract_one(uvec, s, wslots[par * QUAD + s], uo_v, jcbase + s)
                extract_one(ivec, s, hslots[par * QUAD + s], vo_v, jcbase + s)

        NB = CHUNK // QUAD
        for ch in range(BPW // CHUNK):
            chbase = ch * CHUNK
            fire_batch(chbase, 0)

            def body(p, _):
                b0 = chbase + p * (2 * QUAD)
                jc0 = p * (2 * QUAD)
                fire_batch(b0 + QUAD, 1)
                drain_batch(b0, 0)
                extract_batch(b0, 0, jc0)
                fire_batch(b0 + 2 * QUAD, 0)
                drain_batch(b0 + QUAD, 1)
                extract_batch(b0 + QUAD, 1, jc0 + QUAD)
                return 0

            lax.fori_loop(0, NB // 2 - 1, body, 0, unroll=False)
            e0 = chbase + (NB - 2) * QUAD
            fire_batch(e0 + QUAD, 1)
            drain_batch(e0, 0)
            extract_batch(e0, 0, (NB - 2) * QUAD)
            drain_batch(e0 + QUAD, 1)
            extract_batch(e0 + QUAD, 1, (NB - 1) * QUAD)
            pltpu.sync_copy(uo_v, ut_hbm.at[:, pl.ds(base + chbase, CHUNK)])
            pltpu.sync_copy(vo_v, vt_hbm.at[:, pl.ds(base + chbase, CHUNK)])

        for cp in bcps:
            cp.wait()
        pltpu.sync_copy(ubg_v, ubo_hbm.at[pl.ds(base, BPW)])
        pltpu.sync_copy(ibg_v, ibo_hbm.at[pl.ds(base, BPW)])

    return gather_kernel(uidx, iidx, Wt, Ht, ubf, ibf)


def _flat_body(u_ref, i_ref, uo_ref, io_ref):
    uo_ref[...] = u_ref[0, :]
    io_ref[...] = i_ref[0, :]


def _tc_flatten(ubT, ibT):
    n = ubT.shape[1]
    blk = 131072
    grid = (pl.cdiv(n, blk),)
    return pl.pallas_call(
        _flat_body,
        grid=grid,
        in_specs=[
            pl.BlockSpec((1, blk), lambda i: (0, i)),
            pl.BlockSpec((1, blk), lambda i: (0, i)),
        ],
        out_specs=[
            pl.BlockSpec((blk,), lambda i: (i,)),
            pl.BlockSpec((blk,), lambda i: (i,)),
        ],
        out_shape=[
            jax.ShapeDtypeStruct((n,), jnp.float32),
            jax.ShapeDtypeStruct((n,), jnp.float32),
        ],
    )(ubT, ibT)


def _mlp_body(ut_ref, vt_ref, ub_ref, ib_ref, w1a_ref, w1b_ref, b1_ref,
              w2_ref, o_ref):
    h = (
        jnp.dot(w1a_ref[...], ut_ref[...], preferred_element_type=jnp.float32)
        + jnp.dot(w1b_ref[...], vt_ref[...], preferred_element_type=jnp.float32)
        + b1_ref[...]
    )
    h = jnp.maximum(h, 0.0)
    acc = jnp.sum(h * w2_ref[...], axis=0, keepdims=True)
    o_ref[...] = acc + ub_ref[...] + ib_ref[...]


def _tc_mlp(Ut, Vt, ubg, ibg, w1a, w1b, b1c, w2c):
    blk = 2048
    grid = (B // blk,)
    return pl.pallas_call(
        _mlp_body,
        grid=grid,
        in_specs=[
            pl.BlockSpec((K, blk), lambda i: (0, i)),
            pl.BlockSpec((K, blk), lambda i: (0, i)),
            pl.BlockSpec((1, blk), lambda i: (0, i)),
            pl.BlockSpec((1, blk), lambda i: (0, i)),
            pl.BlockSpec((K, K), lambda i: (0, 0)),
            pl.BlockSpec((K, K), lambda i: (0, 0)),
            pl.BlockSpec((K, 1), lambda i: (0, 0)),
            pl.BlockSpec((K, 1), lambda i: (0, 0)),
        ],
        out_specs=pl.BlockSpec((1, blk), lambda i: (0, i)),
        out_shape=jax.ShapeDtypeStruct((1, B), jnp.float32),
    )(Ut, Vt, ubg, ibg, w1a, w1b, b1c, w2c)


def kernel(x, W, H, W_pre, H_pre, W_eps, H_eps, W1, b1, W2, ub, ib):
    uidx = x[:, 0]
    iidx = x[:, 1]
    ubf, ibf = _tc_flatten(ub.T, ib.T)
    Ut, Vt, ubg, ibg = _sc_gather(uidx, iidx, W.T, H.T, ubf, ibf)
    w1a = W1[:, :K]
    w1b = W1[:, K:]
    outT = _tc_mlp(Ut, Vt, ubg.reshape(1, B), ibg.reshape(1, B),
                   w1a, w1b, b1.reshape(K, 1), W2.reshape(K, 1))
    return outT.reshape(B, 1)


# speedup vs baseline: 3.7788x; 1.0146x over previous
import functools

import jax
import jax.numpy as jnp
from jax import lax
from jax.experimental import pallas as pl
from jax.experimental.pallas import tpu as pltpu
from jax.experimental.pallas import tpu_sc as plsc

B = 16384
K = 32
NC = 2
NS = 16
NW = NC * NS
BPW = B // NW
CHUNK = 256
QUAD = 4


def _sc_gather(uidx, iidx, Wt, Ht, ubf, ibf):
    mesh = plsc.VectorSubcoreMesh(core_axis_name="c", subcore_axis_name="s")

    @functools.partial(
        pl.kernel,
        out_type=(
            jax.ShapeDtypeStruct((K, B), jnp.float32),
            jax.ShapeDtypeStruct((K, B), jnp.float32),
            jax.ShapeDtypeStruct((B,), jnp.float32),
            jax.ShapeDtypeStruct((B,), jnp.float32),
        ),
        mesh=mesh,
        scratch_types=[
            pltpu.VMEM((BPW + 16,), jnp.int32),
            pltpu.VMEM((BPW + 16,), jnp.int32),
            [pltpu.VMEM((K, 128), jnp.float32) for _ in range(2 * QUAD)],
            [pltpu.VMEM((K, 128), jnp.float32) for _ in range(2 * QUAD)],
            pltpu.VMEM((K, CHUNK), jnp.float32),
            pltpu.VMEM((K, CHUNK), jnp.float32),
            pltpu.VMEM((BPW,), jnp.float32),
            pltpu.VMEM((BPW,), jnp.float32),
            [pltpu.SemaphoreType.DMA for _ in range(4)],
            pltpu.SemaphoreType.DMA,
        ],
        compiler_params=pltpu.CompilerParams(needs_layout_passes=False),
    )
    def gather_kernel(uidx_hbm, iidx_hbm, wt_hbm, ht_hbm, ub_hbm, ib_hbm,
                      ut_hbm, vt_hbm, ubo_hbm, ibo_hbm,
                      uidx_v, iidx_v, wslots, hslots, uo_v, vo_v,
                      ubg_v, ibg_v, sems, bsem):
        wid = lax.axis_index("s") * NC + lax.axis_index("c")
        base = wid * BPW
        pltpu.sync_copy(uidx_hbm.at[pl.ds(base, BPW)], uidx_v.at[pl.ds(0, BPW)])
        pltpu.sync_copy(iidx_hbm.at[pl.ds(base, BPW)], iidx_v.at[pl.ds(0, BPW)])
        bcps = (
            pltpu.async_copy(ub_hbm.at[uidx_v.at[pl.ds(0, BPW)]], ubg_v, bsem),
            pltpu.async_copy(ib_hbm.at[iidx_v.at[pl.ds(0, BPW)]], ibg_v, bsem),
        )

        klo = lax.iota(jnp.int32, 16)
        khi = klo + 16

        def fire_one(tbl_hbm, idx_vec, lane, slot, sem):
            col = idx_vec[lane]
            off = pl.multiple_of((col >> 7) << 7, 128)
            return pltpu.async_copy(tbl_hbm.at[:, pl.ds(off, 128)], slot, sem)

        def fire_batch(sbase, par):
            uvec = uidx_v[pl.ds(sbase, 16)]
            ivec = iidx_v[pl.ds(sbase, 16)]
            for s in range(QUAD):
                fire_one(wt_hbm, uvec, s, wslots[par * QUAD + s], sems[par])
                fire_one(ht_hbm, ivec, s, hslots[par * QUAD + s], sems[2 + par])

        def drain_batch(sbase, par):
            for s in range(QUAD):
                pltpu.make_async_copy(
                    wt_hbm.at[:, pl.ds(0, 128)], wslots[par * QUAD + s],
                    sems[par]).wait()
                pltpu.make_async_copy(
                    ht_hbm.at[:, pl.ds(0, 128)], hslots[par * QUAD + s],
                    sems[2 + par]).wait()

        def extract_one(idx_vec, lane, slot, out_ref, jc):
            col = idx_vec[lane]
            c = jnp.broadcast_to(col & 127, (16,))
            jcv = jnp.broadcast_to(jc, (16,))
            lo = plsc.load_gather(slot, [klo, c])
            hi = plsc.load_gather(slot, [khi, c])
            plsc.store_scatter(out_ref, [klo, jcv], lo)
            plsc.store_scatter(out_ref, [khi, jcv], hi)

        def extract_batch(sbase, par, jcbase):
            uvec = uidx_v[pl.ds(sbase, 16)]
            ivec = iidx_v[pl.ds(sbase, 16)]
            for s in range(QUAD):
                extract_one(uvec, s, wslots[par * QUAD + s], uo_v, jcbase + s)
                extract_one(ivec, s, hslots[par * QUAD + s], vo_v, jcbase + s)

        NB = CHUNK // QUAD
        for ch in range(BPW // CHUNK):
            chbase = ch * CHUNK
            fire_batch(chbase, 0)

            def body(p, _):
                b0 = chbase + p * (2 * QUAD)
                jc0 = p * (2 * QUAD)
                fire_batch(b0 + QUAD, 1)
                drain_batch(b0, 0)
                extract_batch(b0, 0, jc0)
                fire_batch(b0 + 2 * QUAD, 0)
                drain_batch(b0 + QUAD, 1)
                extract_batch(b0 + QUAD, 1, jc0 + QUAD)
                return 0

            lax.fori_loop(0, NB // 2 - 1, body, 0, unroll=False)
            e0 = chbase + (NB - 2) * QUAD
            fire_batch(e0 + QUAD, 1)
            drain_batch(e0, 0)
            extract_batch(e0, 0, (NB - 2) * QUAD)
            drain_batch(e0 + QUAD, 1)
            extract_batch(e0 + QUAD, 1, (NB - 1) * QUAD)
            pltpu.sync_copy(uo_v, ut_hbm.at[:, pl.ds(base + chbase, CHUNK)])
            pltpu.sync_copy(vo_v, vt_hbm.at[:, pl.ds(base + chbase, CHUNK)])

        for cp in bcps:
            cp.wait()
        pltpu.sync_copy(ubg_v, ubo_hbm.at[pl.ds(base, BPW)])
        pltpu.sync_copy(ibg_v, ibo_hbm.at[pl.ds(base, BPW)])

    return gather_kernel(uidx, iidx, Wt, Ht, ubf, ibf)


def _flat_body(u_ref, i_ref, uo_ref, io_ref):
    uo_ref[...] = u_ref[0, :]
    io_ref[...] = i_ref[0, :]


def _tc_flatten(ubT, ibT):
    n = ubT.shape[1]
    blk = 131072
    grid = (pl.cdiv(n, blk),)
    return pl.pallas_call(
        _flat_body,
        grid=grid,
        in_specs=[
            pl.BlockSpec((1, blk), lambda i: (0, i)),
            pl.BlockSpec((1, blk), lambda i: (0, i)),
        ],
        out_specs=[
            pl.BlockSpec((blk,), lambda i: (i,)),
            pl.BlockSpec((blk,), lambda i: (i,)),
        ],
        out_shape=[
            jax.ShapeDtypeStruct((n,), jnp.float32),
            jax.ShapeDtypeStruct((n,), jnp.float32),
        ],
    )(ubT, ibT)


def _mlp_body(ut_ref, vt_ref, ub_ref, ib_ref, w1a_ref, w1b_ref, b1_ref,
              w2_ref, o_ref):
    h = (
        jnp.dot(w1a_ref[...], ut_ref[...], preferred_element_type=jnp.float32)
        + jnp.dot(w1b_ref[...], vt_ref[...], preferred_element_type=jnp.float32)
        + b1_ref[...]
    )
    h = jnp.maximum(h, 0.0)
    acc = jnp.sum(h * w2_ref[...], axis=0, keepdims=True)
    o_ref[...] = acc + ub_ref[...] + ib_ref[...]


def _tc_mlp(Ut, Vt, ubg, ibg, w1a, w1b, b1c, w2c):
    blk = 2048
    grid = (B // blk,)
    return pl.pallas_call(
        _mlp_body,
        grid=grid,
        in_specs=[
            pl.BlockSpec((K, blk), lambda i: (0, i)),
            pl.BlockSpec((K, blk), lambda i: (0, i)),
            pl.BlockSpec((1, blk), lambda i: (0, i)),
            pl.BlockSpec((1, blk), lambda i: (0, i)),
            pl.BlockSpec((K, K), lambda i: (0, 0)),
            pl.BlockSpec((K, K), lambda i: (0, 0)),
            pl.BlockSpec((K, 1), lambda i: (0, 0)),
            pl.BlockSpec((K, 1), lambda i: (0, 0)),
        ],
        out_specs=pl.BlockSpec((1, blk), lambda i: (0, i)),
        out_shape=jax.ShapeDtypeStruct((1, B), jnp.float32),
    )(Ut, Vt, ubg, ibg, w1a, w1b, b1c, w2c)


def kernel(x, W, H, W_pre, H_pre, W_eps, H_eps, W1, b1, W2, ub, ib):
    uidx = x[:, 0]
    iidx = x[:, 1]
    ubf, ibf = _tc_flatten(ub.T, ib.T)
    Ut, Vt, ubg, ibg = _sc_gather(uidx, iidx, W.T, H.T, ubf, ibf)
    w1a = W1[:, :K]
    w1b = W1[:, K:]
    outT = _tc_mlp(Ut, Vt, ubg.reshape(1, B), ibg.reshape(1, B),
                   w1a, w1b, b1.reshape(K, 1), W2.reshape(K, 1))
    return outT.reshape(B, 1)


# speedup vs baseline: 4.0800x; 1.0797x over previous
import functools

import jax
import jax.numpy as jnp
from jax import lax
from jax.experimental import pallas as pl
from jax.experimental.pallas import tpu as pltpu
from jax.experimental.pallas import tpu_sc as plsc

B = 16384
K = 32
NC = 2
NS = 16
NW = NC * NS
BPW = B // NW
CHUNK = 256
QUAD = 4
NPAR = 3


def _sc_gather(uidx, iidx, Wt, Ht, ubf, ibf):
    mesh = plsc.VectorSubcoreMesh(core_axis_name="c", subcore_axis_name="s")

    @functools.partial(
        pl.kernel,
        out_type=(
            jax.ShapeDtypeStruct((K, B), jnp.float32),
            jax.ShapeDtypeStruct((K, B), jnp.float32),
            jax.ShapeDtypeStruct((B,), jnp.float32),
            jax.ShapeDtypeStruct((B,), jnp.float32),
        ),
        mesh=mesh,
        scratch_types=[
            pltpu.VMEM((BPW + 16,), jnp.int32),
            pltpu.VMEM((BPW + 16,), jnp.int32),
            [pltpu.VMEM((K, 128), jnp.float32) for _ in range(NPAR * QUAD)],
            [pltpu.VMEM((K, 128), jnp.float32) for _ in range(NPAR * QUAD)],
            pltpu.VMEM((K, CHUNK), jnp.float32),
            pltpu.VMEM((K, CHUNK), jnp.float32),
            pltpu.VMEM((BPW,), jnp.float32),
            pltpu.VMEM((BPW,), jnp.float32),
            [pltpu.SemaphoreType.DMA for _ in range(2 * NPAR)],
            pltpu.SemaphoreType.DMA,
        ],
        compiler_params=pltpu.CompilerParams(needs_layout_passes=False),
    )
    def gather_kernel(uidx_hbm, iidx_hbm, wt_hbm, ht_hbm, ub_hbm, ib_hbm,
                      ut_hbm, vt_hbm, ubo_hbm, ibo_hbm,
                      uidx_v, iidx_v, wslots, hslots, uo_v, vo_v,
                      ubg_v, ibg_v, sems, bsem):
        wid = lax.axis_index("s") * NC + lax.axis_index("c")
        base = wid * BPW
        pltpu.sync_copy(uidx_hbm.at[pl.ds(base, BPW)], uidx_v.at[pl.ds(0, BPW)])
        pltpu.sync_copy(iidx_hbm.at[pl.ds(base, BPW)], iidx_v.at[pl.ds(0, BPW)])
        bcps = (
            pltpu.async_copy(ub_hbm.at[uidx_v.at[pl.ds(0, BPW)]], ubg_v, bsem),
            pltpu.async_copy(ib_hbm.at[iidx_v.at[pl.ds(0, BPW)]], ibg_v, bsem),
        )

        klo = lax.iota(jnp.int32, 16)
        khi = klo + 16

        def fire_one(tbl_hbm, idx_vec, lane, slot, sem):
            col = idx_vec[lane]
            off = pl.multiple_of((col >> 7) << 7, 128)
            return pltpu.async_copy(tbl_hbm.at[:, pl.ds(off, 128)], slot, sem)

        def fire_batch(sbase, par):
            uvec = uidx_v[pl.ds(sbase, 16)]
            ivec = iidx_v[pl.ds(sbase, 16)]
            for s in range(QUAD):
                fire_one(wt_hbm, uvec, s, wslots[par * QUAD + s], sems[par])
                fire_one(ht_hbm, ivec, s, hslots[par * QUAD + s],
                         sems[NPAR + par])

        def drain_batch(par):
            for s in range(QUAD):
                pltpu.make_async_copy(
                    wt_hbm.at[:, pl.ds(0, 128)], wslots[par * QUAD + s],
                    sems[par]).wait()
                pltpu.make_async_copy(
                    ht_hbm.at[:, pl.ds(0, 128)], hslots[par * QUAD + s],
                    sems[NPAR + par]).wait()

        def extract_one(idx_vec, lane, slot, out_ref, jc):
            col = idx_vec[lane]
            c = jnp.broadcast_to(col & 127, (16,))
            jcv = jnp.broadcast_to(jc, (16,))
            lo = plsc.load_gather(slot, [klo, c])
            hi = plsc.load_gather(slot, [khi, c])
            plsc.store_scatter(out_ref, [klo, jcv], lo)
            plsc.store_scatter(out_ref, [khi, jcv], hi)

        def extract_batch(sbase, par, jcbase):
            uvec = uidx_v[pl.ds(sbase, 16)]
            ivec = iidx_v[pl.ds(sbase, 16)]
            for s in range(QUAD):
                extract_one(uvec, s, wslots[par * QUAD + s], uo_v, jcbase + s)
                extract_one(ivec, s, hslots[par * QUAD + s], vo_v, jcbase + s)

        NB = CHUNK // QUAD
        STEADY = ((NB - NPAR + 1) // NPAR) * NPAR
        for ch in range(BPW // CHUNK):
            chbase = ch * CHUNK
            fire_batch(chbase, 0)
            fire_batch(chbase + QUAD, 1)

            def step(k, jc):
                par = k % NPAR
                fire_batch(chbase + jc + (NPAR - 1) * QUAD, (k + 2) % NPAR)
                drain_batch(par)
                extract_batch(chbase + jc, par, jc)

            def body(p, _):
                jc0 = p * NPAR * QUAD
                for t in range(NPAR):
                    step(t, jc0 + t * QUAD)
                return 0

            lax.fori_loop(0, STEADY // NPAR, body, 0, unroll=False)
            for k in range(STEADY, NB):
                par = k % NPAR
                if k + NPAR - 1 < NB:
                    fire_batch(chbase + (k + NPAR - 1) * QUAD,
                               (k + NPAR - 1) % NPAR)
                drain_batch(par)
                extract_batch(chbase + k * QUAD, par, k * QUAD)
            pltpu.sync_copy(uo_v, ut_hbm.at[:, pl.ds(base + chbase, CHUNK)])
            pltpu.sync_copy(vo_v, vt_hbm.at[:, pl.ds(base + chbase, CHUNK)])

        for cp in bcps:
            cp.wait()
        pltpu.sync_copy(ubg_v, ubo_hbm.at[pl.ds(base, BPW)])
        pltpu.sync_copy(ibg_v, ibo_hbm.at[pl.ds(base, BPW)])

    return gather_kernel(uidx, iidx, Wt, Ht, ubf, ibf)


def _flat_body(u_ref, i_ref, uo_ref, io_ref):
    uo_ref[...] = u_ref[0, :]
    io_ref[...] = i_ref[0, :]


def _tc_flatten(ubT, ibT):
    n = ubT.shape[1]
    blk = 131072
    grid = (pl.cdiv(n, blk),)
    return pl.pallas_call(
        _flat_body,
        grid=grid,
        in_specs=[
            pl.BlockSpec((1, blk), lambda i: (0, i)),
            pl.BlockSpec((1, blk), lambda i: (0, i)),
        ],
        out_specs=[
            pl.BlockSpec((blk,), lambda i: (i,)),
            pl.BlockSpec((blk,), lambda i: (i,)),
        ],
        out_shape=[
            jax.ShapeDtypeStruct((n,), jnp.float32),
            jax.ShapeDtypeStruct((n,), jnp.float32),
        ],
    )(ubT, ibT)


def _mlp_body(ut_ref, vt_ref, ub_ref, ib_ref, w1a_ref, w1b_ref, b1_ref,
              w2_ref, o_ref):
    h = (
        jnp.dot(w1a_ref[...], ut_ref[...], preferred_element_type=jnp.float32)
        + jnp.dot(w1b_ref[...], vt_ref[...], preferred_element_type=jnp.float32)
        + b1_ref[...]
    )
    h = jnp.maximum(h, 0.0)
    acc = jnp.sum(h * w2_ref[...], axis=0, keepdims=True)
    o_ref[...] = acc + ub_ref[...] + ib_ref[...]


def _tc_mlp(Ut, Vt, ubg, ibg, w1a, w1b, b1c, w2c):
    blk = 2048
    grid = (B // blk,)
    return pl.pallas_call(
        _mlp_body,
        grid=grid,
        in_specs=[
            pl.BlockSpec((K, blk), lambda i: (0, i)),
            pl.BlockSpec((K, blk), lambda i: (0, i)),
            pl.BlockSpec((1, blk), lambda i: (0, i)),
            pl.BlockSpec((1, blk), lambda i: (0, i)),
            pl.BlockSpec((K, K), lambda i: (0, 0)),
            pl.BlockSpec((K, K), lambda i: (0, 0)),
            pl.BlockSpec((K, 1), lambda i: (0, 0)),
            pl.BlockSpec((K, 1), lambda i: (0, 0)),
        ],
        out_specs=pl.BlockSpec((1, blk), lambda i: (0, i)),
        out_shape=jax.ShapeDtypeStruct((1, B), jnp.float32),
    )(Ut, Vt, ubg, ibg, w1a, w1b, b1c, w2c)


def kernel(x, W, H, W_pre, H_pre, W_eps, H_eps, W1, b1, W2, ub, ib):
    uidx = x[:, 0]
    iidx = x[:, 1]
    ubf, ibf = _tc_flatten(ub.T, ib.T)
    Ut, Vt, ubg, ibg = _sc_gather(uidx, iidx, W.T, H.T, ubf, ibf)
    w1a = W1[:, :K]
    w1b = W1[:, K:]
    outT = _tc_mlp(Ut, Vt, ubg.reshape(1, B), ibg.reshape(1, B),
                   w1a, w1b, b1.reshape(K, 1), W2.reshape(K, 1))
    return outT.reshape(B, 1)
